# baseline (speedup 1.0000x reference)
import jax
import jax.numpy as jnp
from jax import lax
from jax.experimental import pallas as pl
from jax.experimental.pallas import tpu as pltpu
from jax.experimental.pallas import tpu_sc as plsc

BATCH, SEQ, DIM = 4, 2048, 1024
NC, NS, L = 2, 16, 16
NW = NC * NS
S_PER_W = SEQ // NW
SUB = 16
NSUB = S_PER_W // SUB
NCHUNK = NSUB * BATCH
NBUFX = 5
PF = 2
VECS = DIM // L
CHUNK_VECS = SUB * VECS


def _body(x_hbm, pe_hbm, pos_hbm, out_hbm, idx_v, pe_v, x_v, in_sems,
          out_sems, g_sems):
    wid = lax.axis_index("s") * NC + lax.axis_index("c")
    s0 = wid * S_PER_W
    pltpu.sync_copy(pos_hbm.at[0, pl.ds(s0, S_PER_W)], idx_v)

    def gather(sub):
        return pltpu.async_copy(
            pe_hbm.at[idx_v.at[pl.ds(sub * SUB, SUB)]], pe_v.at[sub % 2],
            g_sems[sub % 2])

    def row0(t):
        return s0 + (t // BATCH) * SUB

    def start_in(t):
        return pltpu.async_copy(
            x_hbm.at[t % BATCH, pl.ds(row0(t), SUB), :],
            x_v.at[t % NBUFX], in_sems[t % NBUFX])

    ins, outs, gs = {}, {}, {}
    gs[0] = gather(0)
    for t in range(PF):
        ins[t] = start_in(t)
    for t in range(NCHUNK):
        if t >= NBUFX - PF:
            outs[t - (NBUFX - PF)].wait()
        if t + PF < NCHUNK:
            ins[t + PF] = start_in(t + PF)
        if t % BATCH == 0:
            sub = t // BATCH
            if sub + 1 < NSUB:
                gs[sub + 1] = gather(sub + 1)
            gs[sub].wait()
        ins[t].wait()

        slot, pb = t % NBUFX, (t // BATCH) % 2

        @plsc.parallel_loop(0, CHUNK_VECS, unroll=16)
        def _(i):
            r = i // VECS
            sl = pl.ds((i % VECS) * L, L)
            plsc.addupdate(x_v.at[slot, r, sl], pe_v[pb, r, sl])

        outs[t] = pltpu.async_copy(
            x_v.at[slot], out_hbm.at[t % BATCH, pl.ds(row0(t), SUB), :],
            out_sems[slot])
    for t in range(NCHUNK - (NBUFX - PF), NCHUNK):
        outs[t].wait()


def kernel(x, pe_table, position_ids):
    mesh = plsc.VectorSubcoreMesh(core_axis_name="c", subcore_axis_name="s")
    f = pl.kernel(
        _body,
        out_type=jax.ShapeDtypeStruct((BATCH, SEQ, DIM), jnp.float32),
        mesh=mesh,
        scratch_types=[
            pltpu.VMEM((S_PER_W,), jnp.int32),
            pltpu.VMEM((2, SUB, DIM), jnp.float32),
            pltpu.VMEM((NBUFX, SUB, DIM), jnp.float32),
            [pltpu.SemaphoreType.DMA] * NBUFX,
            [pltpu.SemaphoreType.DMA] * NBUFX,
            [pltpu.SemaphoreType.DMA] * 2,
        ],
    )
    return f(x, pe_table, position_ids)


# speedup vs baseline: 1.0196x; 1.0196x over previous
import jax
import jax.numpy as jnp
from jax import lax
from jax.experimental import pallas as pl
from jax.experimental.pallas import tpu as pltpu
from jax.experimental.pallas import tpu_sc as plsc

BATCH, SEQ, DIM = 4, 2048, 1024
NC, NS, L = 2, 16, 16
NW = NC * NS
S_PER_W = SEQ // NW
SUB = 16
NSUB = S_PER_W // SUB
NCHUNK = NSUB * BATCH
NBUFX = 4
PF = 2
VECS = DIM // L
CHUNK_VECS = SUB * VECS


def _body(x_hbm, pe_hbm, pos_hbm, out_hbm, idx_v, pe_v, x_v, in_sems,
          out_sems, g_sems):
    wid = lax.axis_index("s") * NC + lax.axis_index("c")
    s0 = wid * S_PER_W
    pltpu.sync_copy(pos_hbm.at[0, pl.ds(s0, S_PER_W)], idx_v)

    def gather(sub):
        return pltpu.async_copy(
            pe_hbm.at[idx_v.at[pl.ds(sub * SUB, SUB)]], pe_v.at[sub % 2],
            g_sems[sub % 2])

    def row0(t):
        return s0 + (t // BATCH) * SUB

    def start_in(t):
        return pltpu.async_copy(
            x_hbm.at[t % BATCH, pl.ds(row0(t), SUB), :],
            x_v.at[t % NBUFX], in_sems[t % NBUFX])

    ins, outs, gs = {}, {}, {}
    gs[0] = gather(0)
    for t in range(PF):
        ins[t] = start_in(t)
    for t in range(NCHUNK):
        if t >= NBUFX - PF:
            outs[t - (NBUFX - PF)].wait()
        if t + PF < NCHUNK:
            ins[t + PF] = start_in(t + PF)
        if t % BATCH == 0:
            sub = t // BATCH
            if sub + 1 < NSUB:
                gs[sub + 1] = gather(sub + 1)
            gs[sub].wait()
        ins[t].wait()

        slot, pb = t % NBUFX, (t // BATCH) % 2

        @plsc.parallel_loop(0, CHUNK_VECS, unroll=8)
        def _(i):
            r = i // VECS
            sl = pl.ds((i % VECS) * L, L)
            plsc.addupdate(x_v.at[slot, r, sl], pe_v[pb, r, sl])

        outs[t] = pltpu.async_copy(
            x_v.at[slot], out_hbm.at[t % BATCH, pl.ds(row0(t), SUB), :],
            out_sems[slot])
    for t in range(NCHUNK - (NBUFX - PF), NCHUNK):
        outs[t].wait()


def kernel(x, pe_table, position_ids):
    mesh = plsc.VectorSubcoreMesh(core_axis_name="c", subcore_axis_name="s")
    f = pl.kernel(
        _body,
        out_type=jax.ShapeDtypeStruct((BATCH, SEQ, DIM), jnp.float32),
        mesh=mesh,
        scratch_types=[
            pltpu.VMEM((S_PER_W,), jnp.int32),
            pltpu.VMEM((2, SUB, DIM), jnp.float32),
            pltpu.VMEM((NBUFX, SUB, DIM), jnp.float32),
            [pltpu.SemaphoreType.DMA] * NBUFX,
            [pltpu.SemaphoreType.DMA] * NBUFX,
            [pltpu.SemaphoreType.DMA] * 2,
        ],
    )
    return f(x, pe_table, position_ids)


# speedup vs baseline: 1.0204x; 1.0008x over previous
import jax
import jax.numpy as jnp
from jax import lax
from jax.experimental import pallas as pl
from jax.experimental.pallas import tpu as pltpu
from jax.experimental.pallas import tpu_sc as plsc

BATCH, SEQ, DIM = 4, 2048, 1024
NC, NS, L = 2, 16, 16
NW = NC * NS
S_PER_W = SEQ // NW
SUB = 16
NSUB = S_PER_W // SUB
NCHUNK = NSUB * BATCH
NBUFX = 4
PF = 2
VECS = DIM // L
CHUNK_VECS = SUB * VECS


def _body(x_hbm, pe_hbm, pos_hbm, out_hbm, idx_v, pe_v, x_v, in_sems,
          out_sems, g_sems):
    wid = lax.axis_index("s") * NC + lax.axis_index("c")
    s0 = wid * S_PER_W
    pltpu.sync_copy(pos_hbm.at[0, pl.ds(s0, S_PER_W)], idx_v)

    def gather(sub):
        return pltpu.async_copy(
            pe_hbm.at[idx_v.at[pl.ds(sub * SUB, SUB)]], pe_v.at[sub % 2],
            g_sems[sub % 2])

    def row0(t):
        return s0 + (t // BATCH) * SUB

    def start_in(t):
        return pltpu.async_copy(
            x_hbm.at[t % BATCH, pl.ds(row0(t), SUB), :],
            x_v.at[t % NBUFX], in_sems[t % NBUFX])

    ins, outs, gs = {}, {}, {}
    gs[0] = gather(0)
    for t in range(PF):
        ins[t] = start_in(t)
    for t in range(NCHUNK):
        if t >= NBUFX - PF:
            outs[t - (NBUFX - PF)].wait()
        if t + PF < NCHUNK:
            ins[t + PF] = start_in(t + PF)
        if t % BATCH == 0:
            sub = t // BATCH
            if sub + 1 < NSUB:
                gs[sub + 1] = gather(sub + 1)
            gs[sub].wait()
        ins[t].wait()

        slot, pb = t % NBUFX, (t // BATCH) % 2

        @plsc.parallel_loop(0, CHUNK_VECS, unroll=4)
        def _(i):
            r = i // VECS
            sl = pl.ds((i % VECS) * L, L)
            plsc.addupdate(x_v.at[slot, r, sl], pe_v[pb, r, sl])

        outs[t] = pltpu.async_copy(
            x_v.at[slot], out_hbm.at[t % BATCH, pl.ds(row0(t), SUB), :],
            out_sems[slot])
    for t in range(NCHUNK - (NBUFX - PF), NCHUNK):
        outs[t].wait()


def kernel(x, pe_table, position_ids):
    mesh = plsc.VectorSubcoreMesh(core_axis_name="c", subcore_axis_name="s")
    f = pl.kernel(
        _body,
        out_type=jax.ShapeDtypeStruct((BATCH, SEQ, DIM), jnp.float32),
        mesh=mesh,
        scratch_types=[
            pltpu.VMEM((S_PER_W,), jnp.int32),
            pltpu.VMEM((2, SUB, DIM), jnp.float32),
            pltpu.VMEM((NBUFX, SUB, DIM), jnp.float32),
            [pltpu.SemaphoreType.DMA] * NBUFX,
            [pltpu.SemaphoreType.DMA] * NBUFX,
            [pltpu.SemaphoreType.DMA] * 2,
        ],
    )
    return f(x, pe_table, position_ids)
